# Initial kernel scaffold; baseline (speedup 1.0000x reference)
import jax
import jax.numpy as jnp
from jax.experimental import pallas as pl


def kernel(x, edge_index, W1, b1, W2, b2):
    raise NotImplementedError("write your pallas kernel here")


# speedup vs baseline: 5.8565x; 5.8565x over previous
import jax
import jax.numpy as jnp
from jax import lax
from jax.experimental import pallas as pl
from jax.experimental.pallas import tpu as pltpu
from jax.experimental.pallas import tpu_sc as plsc

N = 10000
E = 320000
D = 128
HID = 64

NC = 2
NS = 16
DH = D // NC
EPT = E // NS
B = 80
NB = EPT // B
NPAD = 10240
RPT = NPAD // NS
DW = 16


def _sc_body(xh_hbm, src_hbm, dst_hbm, zsum_hbm, zdeg_hbm, ones_hbm,
             sum_out, deg_out,
             src_v, dst_v, rows_v, ones_v, sum_sh, deg_sh, sem):
    c = lax.axis_index("c")
    s = lax.axis_index("s")

    pltpu.sync_copy(zsum_hbm, sum_sh.at[pl.ds(s * RPT, RPT)])
    pltpu.sync_copy(zdeg_hbm, deg_sh.at[pl.ds(s * RPT, RPT)])
    pltpu.sync_copy(ones_hbm, ones_v)
    pltpu.sync_copy(src_hbm.at[s], src_v)
    pltpu.sync_copy(dst_hbm.at[s], dst_v)
    plsc.subcore_barrier()

    def step(j, carry):
        pltpu.async_copy(xh_hbm.at[c].at[src_v.at[j]], rows_v, sem).wait()
        pltpu.sync_copy(rows_v, sum_sh.at[dst_v.at[j]], add=True)
        pltpu.sync_copy(ones_v, deg_sh.at[dst_v.at[j]], add=True)
        return carry

    lax.fori_loop(0, NB, step, 0)
    plsc.subcore_barrier()

    pltpu.sync_copy(sum_sh.at[pl.ds(s * RPT, RPT)],
                    sum_out.at[c, pl.ds(s * RPT, RPT)])
    pltpu.sync_copy(deg_sh.at[pl.ds(s * RPT, RPT)],
                    deg_out.at[c, pl.ds(s * RPT, RPT)])


def _sc_segment_sum(xh, src_r, dst_r, zsum, zdeg, ones):
    mesh = plsc.VectorSubcoreMesh(core_axis_name="c", subcore_axis_name="s")
    fn = pl.kernel(
        _sc_body,
        out_type=[
            jax.ShapeDtypeStruct((NC, NPAD, DH), jnp.float32),
            jax.ShapeDtypeStruct((NC, NPAD, DW), jnp.float32),
        ],
        mesh=mesh,
        scratch_types=[
            pltpu.VMEM((NB, B), jnp.int32),
            pltpu.VMEM((NB, B), jnp.int32),
            pltpu.VMEM((B, DH), jnp.float32),
            pltpu.VMEM((B, DW), jnp.float32),
            pltpu.VMEM_SHARED((NPAD, DH), jnp.float32),
            pltpu.VMEM_SHARED((NPAD, DW), jnp.float32),
            pltpu.SemaphoreType.DMA,
        ],
        compiler_params=pltpu.CompilerParams(use_tc_tiling_on_sc=False),
        name="sc_segment_sum",
    )
    return fn(xh, src_r, dst_r, zsum, zdeg, ones)


BM = 1000


def _mlp_body(x_ref, sum_ref, deg_ref, w1a_ref, w1b_ref, b1_ref, w2_ref,
              b2_ref, o_ref):
    xb = x_ref[...]
    sb = jnp.concatenate([sum_ref[0], sum_ref[1]], axis=-1)
    dg = deg_ref[0, :, 0:1]
    mean = jnp.where(dg > 0.0, sb / jnp.maximum(dg, 1.0), xb)
    h = jnp.dot(xb, w1a_ref[...], preferred_element_type=jnp.float32)
    h += jnp.dot(mean, w1b_ref[...], preferred_element_type=jnp.float32)
    h = jnp.maximum(h + b1_ref[...], 0.0)
    o_ref[...] = (jnp.dot(h, w2_ref[...], preferred_element_type=jnp.float32)
                  + b2_ref[...])


def _mlp(x, sum_p, deg_p, w1a, w1b, b1, w2, b2):
    return pl.pallas_call(
        _mlp_body,
        grid=(N // BM,),
        in_specs=[
            pl.BlockSpec((BM, D), lambda i: (i, 0)),
            pl.BlockSpec((NC, BM, DH), lambda i: (0, i, 0)),
            pl.BlockSpec((1, BM, DW), lambda i: (0, i, 0)),
            pl.BlockSpec((D, HID), lambda i: (0, 0)),
            pl.BlockSpec((D, HID), lambda i: (0, 0)),
            pl.BlockSpec((1, HID), lambda i: (0, 0)),
            pl.BlockSpec((HID, D), lambda i: (0, 0)),
            pl.BlockSpec((1, D), lambda i: (0, 0)),
        ],
        out_specs=pl.BlockSpec((BM, D), lambda i: (i, 0)),
        out_shape=jax.ShapeDtypeStruct((N, D), jnp.float32),
        name="mlp_mixer",
    )(x, sum_p, deg_p, w1a, w1b, b1, w2, b2)


def kernel(x, edge_index, W1, b1, W2, b2):
    xh = x.reshape(N, NC, DH).transpose(1, 0, 2)
    src_r = edge_index[0].reshape(NS, NB, B)
    dst_r = edge_index[1].reshape(NS, NB, B)
    zsum = jnp.zeros((RPT, DH), jnp.float32)
    zdeg = jnp.zeros((RPT, DW), jnp.float32)
    ones = jnp.ones((B, DW), jnp.float32)
    sum_p, deg_p = _sc_segment_sum(xh, src_r, dst_r, zsum, zdeg, ones)
    return _mlp(x, sum_p, deg_p, W1[:D], W1[D:], b1.reshape(1, HID), W2,
                b2.reshape(1, D))


# speedup vs baseline: 7.4865x; 1.2783x over previous
import jax
import jax.numpy as jnp
from jax import lax
from jax.experimental import pallas as pl
from jax.experimental.pallas import tpu as pltpu
from jax.experimental.pallas import tpu_sc as plsc

N = 10000
E = 320000
D = 128
HID = 64

NC = 2
NS = 16
DH = D // NC
EPT = E // NS
B = 80
NB = EPT // B
NPAD = 10240
RPT = NPAD // NS
DW = 16


def _sc_body(xh_hbm, src_hbm, dst_hbm, zsum_hbm, zdeg_hbm, ones_hbm,
             sum_out, deg_out,
             src_v, dst_v, rv0, rv1, ones_v, sum_sh, deg_sh,
             g0, g1, s0, s1, d0):
    c = lax.axis_index("c")
    s = lax.axis_index("s")

    pltpu.sync_copy(zsum_hbm, sum_sh.at[pl.ds(s * RPT, RPT)])
    pltpu.sync_copy(zdeg_hbm, deg_sh.at[pl.ds(s * RPT, RPT)])
    pltpu.sync_copy(ones_hbm, ones_v)
    pltpu.sync_copy(src_hbm.at[s], src_v)
    pltpu.sync_copy(dst_hbm.at[s], dst_v)
    plsc.subcore_barrier()

    def gstart(j, rv, sem):
        pltpu.async_copy(xh_hbm.at[c].at[src_v.at[j]], rv, sem)

    def gwait(rv, sem):
        pltpu.make_async_copy(xh_hbm.at[c].at[src_v.at[0]], rv, sem).wait()

    def sstart(rv, j, sem):
        pltpu.async_copy(rv, sum_sh.at[dst_v.at[j]], sem, add=True)

    def swait(rv, sem):
        pltpu.make_async_copy(rv, sum_sh.at[dst_v.at[0]], sem).wait()

    gstart(0, rv0, g0)

    def outer(i, carry):
        j0 = 2 * i
        j1 = j0 + 1
        gwait(rv0, g0)

        @pl.when(i > 0)
        def _():
            swait(rv1, s1)
        gstart(j1, rv1, g1)
        sstart(rv0, j0, s0)

        @pl.when((c == 0) & (i > 0))
        def _():
            pltpu.make_async_copy(ones_v, deg_sh.at[dst_v.at[0]], d0).wait()

        @pl.when(c == 0)
        def _():
            pltpu.async_copy(ones_v, deg_sh.at[dst_v.at[j0]], d0, add=True)

        gwait(rv1, g1)
        swait(rv0, s0)
        jn = jnp.minimum(j0 + 2, NB - 1)
        gstart(jn, rv0, g0)
        sstart(rv1, j1, s1)

        @pl.when((c == 1) & (i > 0))
        def _():
            pltpu.make_async_copy(ones_v, deg_sh.at[dst_v.at[0]], d0).wait()

        @pl.when(c == 1)
        def _():
            pltpu.async_copy(ones_v, deg_sh.at[dst_v.at[j1]], d0, add=True)

        return carry

    lax.fori_loop(0, NB // 2, outer, 0)
    gwait(rv0, g0)
    swait(rv1, s1)
    pltpu.make_async_copy(ones_v, deg_sh.at[dst_v.at[0]], d0).wait()
    plsc.subcore_barrier()

    pltpu.sync_copy(sum_sh.at[pl.ds(s * RPT, RPT)],
                    sum_out.at[c, pl.ds(s * RPT, RPT)])
    pltpu.sync_copy(deg_sh.at[pl.ds(s * RPT, RPT)],
                    deg_out.at[c, pl.ds(s * RPT, RPT)])


def _sc_segment_sum(xh, src_r, dst_r, zsum, zdeg, ones):
    mesh = plsc.VectorSubcoreMesh(core_axis_name="c", subcore_axis_name="s")
    fn = pl.kernel(
        _sc_body,
        out_type=[
            jax.ShapeDtypeStruct((NC, NPAD, DH), jnp.float32),
            jax.ShapeDtypeStruct((NC, NPAD, DW), jnp.float32),
        ],
        mesh=mesh,
        scratch_types=[
            pltpu.VMEM((NB, B), jnp.int32),
            pltpu.VMEM((NB, B), jnp.int32),
            pltpu.VMEM((B, DH), jnp.float32),
            pltpu.VMEM((B, DH), jnp.float32),
            pltpu.VMEM((B, DW), jnp.float32),
            pltpu.VMEM_SHARED((NPAD, DH), jnp.float32),
            pltpu.VMEM_SHARED((NPAD, DW), jnp.float32),
            pltpu.SemaphoreType.DMA,
            pltpu.SemaphoreType.DMA,
            pltpu.SemaphoreType.DMA,
            pltpu.SemaphoreType.DMA,
            pltpu.SemaphoreType.DMA,
        ],
        compiler_params=pltpu.CompilerParams(use_tc_tiling_on_sc=False),
        name="sc_segment_sum",
    )
    return fn(xh, src_r, dst_r, zsum, zdeg, ones)


BM = 1000


def _mlp_body(x_ref, sum_ref, deg_ref, w1a_ref, w1b_ref, b1_ref, w2_ref,
              b2_ref, o_ref):
    xb = x_ref[...]
    sb = jnp.concatenate([sum_ref[0], sum_ref[1]], axis=-1)
    dg = deg_ref[0, :, 0:1] + deg_ref[1, :, 0:1]
    mean = jnp.where(dg > 0.0, sb / jnp.maximum(dg, 1.0), xb)
    h = jnp.dot(xb, w1a_ref[...], preferred_element_type=jnp.float32)
    h += jnp.dot(mean, w1b_ref[...], preferred_element_type=jnp.float32)
    h = jnp.maximum(h + b1_ref[...], 0.0)
    o_ref[...] = (jnp.dot(h, w2_ref[...], preferred_element_type=jnp.float32)
                  + b2_ref[...])


def _mlp(x, sum_p, deg_p, w1a, w1b, b1, w2, b2):
    return pl.pallas_call(
        _mlp_body,
        grid=(N // BM,),
        in_specs=[
            pl.BlockSpec((BM, D), lambda i: (i, 0)),
            pl.BlockSpec((NC, BM, DH), lambda i: (0, i, 0)),
            pl.BlockSpec((NC, BM, DW), lambda i: (0, i, 0)),
            pl.BlockSpec((D, HID), lambda i: (0, 0)),
            pl.BlockSpec((D, HID), lambda i: (0, 0)),
            pl.BlockSpec((1, HID), lambda i: (0, 0)),
            pl.BlockSpec((HID, D), lambda i: (0, 0)),
            pl.BlockSpec((1, D), lambda i: (0, 0)),
        ],
        out_specs=pl.BlockSpec((BM, D), lambda i: (i, 0)),
        out_shape=jax.ShapeDtypeStruct((N, D), jnp.float32),
        name="mlp_mixer",
    )(x, sum_p, deg_p, w1a, w1b, b1, w2, b2)


def kernel(x, edge_index, W1, b1, W2, b2):
    xh = x.reshape(N, NC, DH).transpose(1, 0, 2)
    src_r = edge_index[0].reshape(NS, NB, B)
    dst_r = edge_index[1].reshape(NS, NB, B)
    zsum = jnp.zeros((RPT, DH), jnp.float32)
    zdeg = jnp.zeros((RPT, DW), jnp.float32)
    ones = jnp.ones((B, DW), jnp.float32)
    sum_p, deg_p = _sc_segment_sum(xh, src_r, dst_r, zsum, zdeg, ones)
    return _mlp(x, sum_p, deg_p, W1[:D], W1[D:], b1.reshape(1, HID), W2,
                b2.reshape(1, D))


# speedup vs baseline: 13.9459x; 1.8628x over previous
import jax
import jax.numpy as jnp
from jax import lax
from jax.experimental import pallas as pl
from jax.experimental.pallas import tpu as pltpu
from jax.experimental.pallas import tpu_sc as plsc

N = 10000
E = 320000
D = 128
HID = 64

NC = 2
NS = 16
NW = NC * NS
EPT = E // NW
B = 40
NB = EPT // B
NBUF = 4
RPT = 624
RLAST = N - 15 * RPT
DW = 8


def _sc_body(x_hbm, src_hbm, dst_hbm, zsum_hbm, zdeg_hbm, ones_hbm,
             sum_out, deg_out,
             src_v, dst_v, rv0, rv1, rv2, rv3, ones_v, sum_sh, deg_sh,
             g0, g1, g2, g3, s0, s1, s2, s3, d0):
    c = lax.axis_index("c")
    s = lax.axis_index("s")
    w = c * NS + s
    rvs = [rv0, rv1, rv2, rv3]
    gs = [g0, g1, g2, g3]
    ss = [s0, s1, s2, s3]

    @pl.when(s < NS - 1)
    def _():
        pltpu.sync_copy(zsum_hbm.at[pl.ds(0, RPT)],
                        sum_sh.at[pl.ds(s * RPT, RPT)])
        pltpu.sync_copy(zdeg_hbm.at[pl.ds(0, RPT)],
                        deg_sh.at[pl.ds(s * RPT, RPT)])

    @pl.when(s == NS - 1)
    def _():
        pltpu.sync_copy(zsum_hbm, sum_sh.at[pl.ds(N - RLAST, RLAST)])
        pltpu.sync_copy(zdeg_hbm, deg_sh.at[pl.ds(N - RLAST, RLAST)])

    pltpu.sync_copy(ones_hbm, ones_v)
    pltpu.sync_copy(src_hbm.at[w], src_v)
    pltpu.sync_copy(dst_hbm.at[w], dst_v)
    plsc.subcore_barrier()

    def gstart(j, b):
        pltpu.async_copy(x_hbm.at[src_v.at[j]], rvs[b], gs[b])

    def gwait(b):
        pltpu.make_async_copy(x_hbm.at[src_v.at[0]], rvs[b], gs[b]).wait()

    def sstart(j, b):
        pltpu.async_copy(rvs[b], sum_sh.at[dst_v.at[j]], ss[b], add=True)

    def swait(b):
        pltpu.make_async_copy(rvs[b], sum_sh.at[dst_v.at[0]], ss[b]).wait()

    def dstart(j):
        pltpu.async_copy(ones_v, deg_sh.at[dst_v.at[j]], d0, add=True)

    def dwait():
        pltpu.make_async_copy(ones_v, deg_sh.at[dst_v.at[0]], d0).wait()

    for b in range(NBUF - 1):
        gstart(b, b)

    def group(g, carry):
        for b in range(NBUF):
            j = g * NBUF + b
            gwait(b)
            sstart(j, b)
            if b == 0:
                @pl.when(g > 0)
                def _():
                    swait(NBUF - 1)
                    dwait()
            else:
                swait(b - 1)
                dwait()
            dstart(j)
            gstart(jnp.minimum(j + NBUF - 1, NB - 1), (b + NBUF - 1) % NBUF)
        return carry

    NG = (NB - 1) // NBUF
    lax.fori_loop(0, NG, group, 0)

    for j in range(NG * NBUF, NB):
        b = j % NBUF
        gwait(b)
        sstart(j, b)
        swait((j - 1) % NBUF)
        dwait()
        dstart(j)

    swait((NB - 1) % NBUF)
    dwait()
    for k in range((NBUF - 1) + NG * NBUF - NB):
        gwait((NB + k) % NBUF)
    plsc.subcore_barrier()

    @pl.when(s < NS - 1)
    def _():
        pltpu.sync_copy(sum_sh.at[pl.ds(s * RPT, RPT)],
                        sum_out.at[c].at[pl.ds(s * RPT, RPT)])
        pltpu.sync_copy(deg_sh.at[pl.ds(s * RPT, RPT)],
                        deg_out.at[c].at[pl.ds(s * RPT, RPT)])

    @pl.when(s == NS - 1)
    def _():
        pltpu.sync_copy(sum_sh.at[pl.ds(N - RLAST, RLAST)],
                        sum_out.at[c].at[pl.ds(N - RLAST, RLAST)])
        pltpu.sync_copy(deg_sh.at[pl.ds(N - RLAST, RLAST)],
                        deg_out.at[c].at[pl.ds(N - RLAST, RLAST)])


def _sc_segment_sum(x, src_r, dst_r, zsum, zdeg, ones):
    mesh = plsc.VectorSubcoreMesh(core_axis_name="c", subcore_axis_name="s")
    fn = pl.kernel(
        _sc_body,
        out_type=[
            jax.ShapeDtypeStruct((NC, N, D), jnp.float32),
            jax.ShapeDtypeStruct((NC, N, DW), jnp.float32),
        ],
        mesh=mesh,
        scratch_types=[
            pltpu.VMEM((NB, B), jnp.int32),
            pltpu.VMEM((NB, B), jnp.int32),
            pltpu.VMEM((B, D), jnp.float32),
            pltpu.VMEM((B, D), jnp.float32),
            pltpu.VMEM((B, D), jnp.float32),
            pltpu.VMEM((B, D), jnp.float32),
            pltpu.VMEM((B, DW), jnp.float32),
            pltpu.VMEM_SHARED((N, D), jnp.float32),
            pltpu.VMEM_SHARED((N, DW), jnp.float32),
            pltpu.SemaphoreType.DMA,
            pltpu.SemaphoreType.DMA,
            pltpu.SemaphoreType.DMA,
            pltpu.SemaphoreType.DMA,
            pltpu.SemaphoreType.DMA,
            pltpu.SemaphoreType.DMA,
            pltpu.SemaphoreType.DMA,
            pltpu.SemaphoreType.DMA,
            pltpu.SemaphoreType.DMA,
        ],
        compiler_params=pltpu.CompilerParams(use_tc_tiling_on_sc=False),
        name="sc_segment_sum",
    )
    return fn(x, src_r, dst_r, zsum, zdeg, ones)


BM = 1000


def _mlp_body(x_ref, sum_ref, deg_ref, w1a_ref, w1b_ref, b1_ref, w2_ref,
              b2_ref, o_ref):
    xb = x_ref[...]
    sb = sum_ref[0] + sum_ref[1]
    dg = deg_ref[0, :, 0:1] + deg_ref[1, :, 0:1]
    mean = jnp.where(dg > 0.0, sb / jnp.maximum(dg, 1.0), xb)
    h = jnp.dot(xb, w1a_ref[...], preferred_element_type=jnp.float32)
    h += jnp.dot(mean, w1b_ref[...], preferred_element_type=jnp.float32)
    h = jnp.maximum(h + b1_ref[...], 0.0)
    o_ref[...] = (jnp.dot(h, w2_ref[...], preferred_element_type=jnp.float32)
                  + b2_ref[...])


def _mlp(x, sum_p, deg_p, w1a, w1b, b1, w2, b2):
    return pl.pallas_call(
        _mlp_body,
        grid=(N // BM,),
        in_specs=[
            pl.BlockSpec((BM, D), lambda i: (i, 0)),
            pl.BlockSpec((NC, BM, D), lambda i: (0, i, 0)),
            pl.BlockSpec((NC, BM, DW), lambda i: (0, i, 0)),
            pl.BlockSpec((D, HID), lambda i: (0, 0)),
            pl.BlockSpec((D, HID), lambda i: (0, 0)),
            pl.BlockSpec((1, HID), lambda i: (0, 0)),
            pl.BlockSpec((HID, D), lambda i: (0, 0)),
            pl.BlockSpec((1, D), lambda i: (0, 0)),
        ],
        out_specs=pl.BlockSpec((BM, D), lambda i: (i, 0)),
        out_shape=jax.ShapeDtypeStruct((N, D), jnp.float32),
        name="mlp_mixer",
    )(x, sum_p, deg_p, w1a, w1b, b1, w2, b2)


def kernel(x, edge_index, W1, b1, W2, b2):
    src_r = edge_index[0].reshape(NW, NB, B)
    dst_r = edge_index[1].reshape(NW, NB, B)
    zsum = jnp.zeros((RLAST, D), jnp.float32)
    zdeg = jnp.zeros((RLAST, DW), jnp.float32)
    ones = jnp.ones((B, DW), jnp.float32)
    sum_p, deg_p = _sc_segment_sum(x, src_r, dst_r, zsum, zdeg, ones)
    return _mlp(x, sum_p, deg_p, W1[:D], W1[D:], b1.reshape(1, HID), W2,
                b2.reshape(1, D))
